# baseline (speedup 1.0000x reference)
import jax
import jax.numpy as jnp
from jax import lax
from jax.experimental import pallas as pl
from jax.experimental.pallas import tpu as pltpu
from jax.experimental.pallas import tpu_sc as plsc

BATCH = 16384
HIST = 50
EMBED_DIM = 64
B = BATCH * HIST
NC, NS = 2, 16
NW = NC * NS
B_PER_W = B // NW
CHUNK = 512
N_CHUNKS = B_PER_W // CHUNK
NBUF = 2


NSPLIT = 4
SUB = CHUNK // NSPLIT


def _gather_kernel(idx_hbm, table_hbm, out_hbm, idx_v, rows_v,
                   sem_g0, sem_g1, sem_o0, sem_o1):
    wid = lax.axis_index("s") * NC + lax.axis_index("c")
    base = wid * B_PER_W
    sem_g = (sem_g0, sem_g1)
    sem_o = (sem_o0, sem_o1)

    pltpu.sync_copy(idx_hbm.at[pl.ds(base, B_PER_W)], idx_v)

    def start_gather(i, b):
        for k in range(NSPLIT):
            pltpu.async_copy(
                table_hbm.at[idx_v.at[pl.ds(i * CHUNK + k * SUB, SUB)]],
                rows_v.at[b].at[pl.ds(k * SUB, SUB)], sem_g[b])

    def wait_gather(i, b):
        for k in range(NSPLIT):
            pltpu.make_async_copy(
                table_hbm.at[idx_v.at[pl.ds(i * CHUNK + k * SUB, SUB)]],
                rows_v.at[b].at[pl.ds(k * SUB, SUB)], sem_g[b]).wait()

    def start_out(i, b):
        pltpu.async_copy(
            rows_v.at[b], out_hbm.at[pl.ds(base + i * CHUNK, CHUNK)],
            sem_o[b])

    def wait_out(i, b):
        pltpu.make_async_copy(
            rows_v.at[b], out_hbm.at[pl.ds(base + i * CHUNK, CHUNK)],
            sem_o[b]).wait()

    start_gather(0, 0)
    start_gather(1, 1)
    wait_gather(0, 0)
    start_out(0, 0)

    def body(j, carry):
        for b in range(NBUF):
            i = 2 + j * NBUF + b
            wait_out(i - 2, b)
            start_gather(i, b)
            wait_gather(i - 1, (b + 1) % NBUF)
            start_out(i - 1, (b + 1) % NBUF)
        return carry

    lax.fori_loop(0, (N_CHUNKS - 2) // NBUF, body, 0)

    last = N_CHUNKS - 1
    wait_gather(last, last % NBUF)
    start_out(last, last % NBUF)
    wait_out(last - 1, (last - 1) % NBUF)
    wait_out(last, last % NBUF)


def kernel(data, embedding_table):
    idx = data.reshape(B)
    mesh = plsc.VectorSubcoreMesh(core_axis_name="c", subcore_axis_name="s")
    out = pl.kernel(
        _gather_kernel,
        out_type=jax.ShapeDtypeStruct((B, EMBED_DIM), jnp.float32),
        mesh=mesh,
        scratch_types=[
            pltpu.VMEM((B_PER_W,), jnp.int32),
            pltpu.VMEM((NBUF, CHUNK, EMBED_DIM), jnp.float32),
            pltpu.SemaphoreType.DMA,
            pltpu.SemaphoreType.DMA,
            pltpu.SemaphoreType.DMA,
            pltpu.SemaphoreType.DMA,
        ],
        compiler_params=pltpu.CompilerParams(use_tc_tiling_on_sc=False),
    )(idx, embedding_table)
    return out.reshape(BATCH, HIST, EMBED_DIM)
